# baseline (speedup 1.0000x reference)
import jax
import jax.numpy as jnp
from jax import lax
from jax.experimental import pallas as pl
from jax.experimental.pallas import tpu as pltpu
from jax.experimental.pallas import tpu_sc as plsc

N = 50000
E = 800000
C = 5000
DEMO = 8

_SC_W = 1000
_CW = 2000
_EPT = E // 16


def _conv_msg_body(g0, g1, g2, g3, m0, m1, m2, m3, src_hbm, dst_hbm,
                   o0, o1, o2, o3,
                   idx_v, dst_v, pay_v, rows_v, acc_sh, sem):
    c = lax.axis_index("c")
    s = lax.axis_index("s")
    zeros16 = jnp.zeros((16,), jnp.float32)

    def run(g_hbm, m_hbm, out_hbm):
        def zf(j, _):
            pay_v[j, pl.ds(0, 16)] = zeros16
            return 0
        lax.fori_loop(0, _CW, zf, 0)
        zbase = s * 3128
        pltpu.sync_copy(pay_v, acc_sh.at[pl.ds(zbase, _CW)])

        @pl.when(s < 15)
        def _():
            pltpu.sync_copy(pay_v.at[pl.ds(0, 1128)],
                            acc_sh.at[pl.ds(zbase + _CW, 1128)])

        @pl.when(s == 15)
        def _():
            pltpu.sync_copy(pay_v.at[pl.ds(0, 1080)],
                            acc_sh.at[pl.ds(zbase + _CW, 1080)])
        plsc.subcore_barrier()

        def win(g, _):
            base = s * _EPT + g * _CW
            pltpu.sync_copy(src_hbm.at[pl.ds(base, _CW)], idx_v)
            pltpu.sync_copy(dst_hbm.at[pl.ds(base, _CW)], dst_v)
            pltpu.sync_copy(m_hbm.at[pl.ds(base, _CW)], pay_v)
            pltpu.async_copy(g_hbm.at[idx_v], rows_v, sem).wait()

            def mul(i, _):
                for k in range(8):
                    r = i * 8 + k
                    rows_v[r, pl.ds(0, 16)] = rows_v[r, pl.ds(0, 16)] * pay_v[r, pl.ds(0, 16)]
                return 0
            lax.fori_loop(0, _CW // 8, mul, 0)
            pltpu.sync_copy(rows_v, acc_sh.at[dst_v], add=True)
            return 0
        lax.fori_loop(0, _EPT // _CW, win, 0)
        plsc.subcore_barrier()

        @pl.when(s < 15)
        def _():
            pltpu.sync_copy(acc_sh.at[pl.ds(s * 3128, 3128)],
                            out_hbm.at[pl.ds(s * 3128, 3128)])

        @pl.when(s == 15)
        def _():
            pltpu.sync_copy(acc_sh.at[pl.ds(15 * 3128, 3080)],
                            out_hbm.at[pl.ds(15 * 3128, 3080)])
        plsc.subcore_barrier()

    gq = (g0, g1, g2, g3)
    mq = (m0, m1, m2, m3)
    oq = (o0, o1, o2, o3)
    for qi in range(4):
        @pl.when(c == qi // 2)
        def _(qi=qi):
            run(gq[qi], mq[qi], oq[qi])


_EPC = E // 2
_EPT2 = _EPC // 16


def _mask_body(asrc_hbm, adst_hbm, ep_hbm, src_hbm, dst_hbm, mp_hbm,
               idx_v, dst_v, a_v, b_v, e_v, sem):
    c = lax.axis_index("c")
    s = lax.axis_index("s")

    def win(g, _):
        base = c * _EPC + s * _EPT2 + g * _SC_W
        pltpu.sync_copy(src_hbm.at[pl.ds(base, _SC_W)], idx_v)
        pltpu.sync_copy(dst_hbm.at[pl.ds(base, _SC_W)], dst_v)
        pltpu.sync_copy(ep_hbm.at[pl.ds(base, _SC_W)], e_v)
        pltpu.async_copy(asrc_hbm.at[idx_v], a_v, sem).wait()
        pltpu.async_copy(adst_hbm.at[dst_v], b_v, sem).wait()

        def add(i, _):
            for k in range(4):
                r = i * 4 + k
                e_v[r, pl.ds(0, 16)] = e_v[r, pl.ds(0, 16)] + a_v[r, pl.ds(0, 16)] + b_v[r, pl.ds(0, 16)]
                e_v[r, pl.ds(16, 16)] = e_v[r, pl.ds(16, 16)] + a_v[r, pl.ds(16, 16)] + b_v[r, pl.ds(16, 16)]
            return 0
        lax.fori_loop(0, _SC_W // 4, add, 0)
        pltpu.sync_copy(e_v, mp_hbm.at[pl.ds(base, _SC_W)])
        return 0
    lax.fori_loop(0, _EPT2 // _SC_W, win, 0)


def _mask_sc(asrc, adst, e_part, src, dst):
    f32 = jnp.float32
    mesh = plsc.VectorSubcoreMesh(core_axis_name="c", subcore_axis_name="s")
    return pl.kernel(
        _mask_body,
        out_type=jax.ShapeDtypeStruct((E, 32), f32),
        mesh=mesh,
        scratch_types=[
            pltpu.VMEM((_SC_W,), jnp.int32),
            pltpu.VMEM((_SC_W,), jnp.int32),
            pltpu.VMEM((_SC_W, 32), f32),
            pltpu.VMEM((_SC_W, 32), f32),
            pltpu.VMEM((_SC_W, 32), f32),
            pltpu.SemaphoreType.DMA,
        ],
        compiler_params=pltpu.CompilerParams(use_tc_tiling_on_sc=False),
    )(asrc, adst, e_part, src, dst)


def _hist_body(dst_hbm, comm_hbm, cnt0_hbm, cnt1_hbm, pc0_hbm, pc1_hbm,
               dst_v, ones_v, cnt_sh, pcnt_sh, sem):
    c = lax.axis_index("c")
    s = lax.axis_index("s")
    ones16 = jnp.ones((16,), jnp.float32)
    zeros16 = jnp.zeros((16,), jnp.float32)

    def fillz(j, _):
        ones_v[j, pl.ds(0, 16)] = zeros16
        return 0
    lax.fori_loop(0, _SC_W, fillz, 0)
    zbase = s * 3128
    pltpu.sync_copy(ones_v, cnt_sh.at[pl.ds(zbase, _SC_W)])
    pltpu.sync_copy(ones_v, cnt_sh.at[pl.ds(zbase + _SC_W, _SC_W)])
    pltpu.sync_copy(ones_v, cnt_sh.at[pl.ds(zbase + 2 * _SC_W, _SC_W)])

    @pl.when(s < 15)
    def _():
        pltpu.sync_copy(ones_v.at[pl.ds(0, 128)],
                        cnt_sh.at[pl.ds(zbase + 3 * _SC_W, 128)])

    @pl.when(s == 15)
    def _():
        pltpu.sync_copy(ones_v.at[pl.ds(0, 80)],
                        cnt_sh.at[pl.ds(zbase + 3 * _SC_W, 80)])

    @pl.when(s < 15)
    def _():
        pltpu.sync_copy(ones_v.at[pl.ds(0, 312)], pcnt_sh.at[pl.ds(s * 312, 312)])

    @pl.when(s == 15)
    def _():
        pltpu.sync_copy(ones_v.at[pl.ds(0, 320)], pcnt_sh.at[pl.ds(4680, 320)])

    def fillo(j, _):
        ones_v[j, pl.ds(0, 16)] = ones16
        return 0
    lax.fori_loop(0, _SC_W, fillo, 0)
    plsc.subcore_barrier()

    def win(g, _):
        base = c * _EPC + s * _EPT2 + g * _SC_W
        pltpu.sync_copy(dst_hbm.at[pl.ds(base, _SC_W)], dst_v)
        pltpu.sync_copy(ones_v, cnt_sh.at[dst_v], add=True)
        return 0
    lax.fori_loop(0, _EPT2 // _SC_W, win, 0)

    for w in range(25):
        @pl.when(s == w % 16)
        def _(w=w):
            pltpu.sync_copy(comm_hbm.at[pl.ds(c * _HNPC + w * _SC_W, _SC_W)], dst_v)
            pltpu.sync_copy(ones_v, pcnt_sh.at[dst_v], add=True)
    plsc.subcore_barrier()

    def writeout(cnt_hbm):
        @pl.when(s < 15)
        def _():
            pltpu.sync_copy(cnt_sh.at[pl.ds(s * 3128, 3128)],
                            cnt_hbm.at[pl.ds(s * 3128, 3128)])

        @pl.when(s == 15)
        def _():
            pltpu.sync_copy(cnt_sh.at[pl.ds(15 * 3128, 3080)],
                            cnt_hbm.at[pl.ds(15 * 3128, 3080)])

    @pl.when(c == 0)
    def _():
        writeout(cnt0_hbm)

        @pl.when(s == 0)
        def _():
            pltpu.sync_copy(pcnt_sh, pc0_hbm)

    @pl.when(c == 1)
    def _():
        writeout(cnt1_hbm)

        @pl.when(s == 0)
        def _():
            pltpu.sync_copy(pcnt_sh, pc1_hbm)


def _hist_sc(dst, community):
    f32 = jnp.float32
    mesh = plsc.VectorSubcoreMesh(core_axis_name="c", subcore_axis_name="s")
    return pl.kernel(
        _hist_body,
        out_type=[jax.ShapeDtypeStruct((N, 16), f32),
                  jax.ShapeDtypeStruct((N, 16), f32),
                  jax.ShapeDtypeStruct((C, 16), f32),
                  jax.ShapeDtypeStruct((C, 16), f32)],
        mesh=mesh,
        scratch_types=[
            pltpu.VMEM((_SC_W,), jnp.int32),
            pltpu.VMEM((_SC_W, 16), f32),
            pltpu.VMEM_SHARED((N, 16), f32),
            pltpu.VMEM_SHARED((C, 16), f32),
            pltpu.SemaphoreType.DMA,
        ],
        compiler_params=pltpu.CompilerParams(use_tc_tiling_on_sc=False),
    )(dst, community)


_HNPC = N // 2
_PQ = (6256, 6256, 6256, 6232)
_PW = 784


def _pool_body(comm_hbm, h0, h1, h2, h3,
               osum0, osum1, osum2, osum3, omax0, omax1, omax2, omax3,
               comm_v, sidx_v, rows_v, max_acc, sum_sh, sem):
    c = lax.axis_index("c")
    s = lax.axis_index("s")
    nsub = s // 4
    cg = s % 4
    hq = (h0, h1, h2, h3)
    osq = (osum0, osum1, osum2, osum3)
    omq = (omax0, omax1, omax2, omax3)
    zeros16 = jnp.zeros((16,), jnp.float32)
    neg16 = jnp.full((16,), -3.0e38, jnp.float32)
    iota16 = lax.iota(jnp.int32, 16)

    def initm(j, _):
        max_acc[j, pl.ds(0, 16)] = neg16
        return 0
    lax.fori_loop(0, C, initm, 0)

    def fillz(j, _):
        rows_v[j, pl.ds(0, 16)] = zeros16
        return 0
    lax.fori_loop(0, _PW, fillz, 0)
    zb = s * 1256
    pltpu.sync_copy(rows_v, sum_sh.at[pl.ds(zb, _PW)])

    @pl.when(s < 15)
    def _():
        pltpu.sync_copy(rows_v.at[pl.ds(0, 472)], sum_sh.at[pl.ds(zb + _PW, 472)])

    @pl.when(s == 15)
    def _():
        pltpu.sync_copy(rows_v.at[pl.ds(0, 376)], sum_sh.at[pl.ds(zb + _PW, 376)])
    plsc.subcore_barrier()

    strip_base = c * _HNPC + nsub * 6256

    def window(wbase, wlen):
        base = strip_base + wbase
        pltpu.sync_copy(comm_hbm.at[pl.ds(base, wlen)], comm_v.at[pl.ds(0, wlen)])
        for qi in range(4):
            @pl.when(cg == qi)
            def _(qi=qi):
                pltpu.sync_copy(hq[qi].at[pl.ds(base, wlen)], rows_v.at[pl.ds(0, wlen)])
        if wlen < _PW:
            def padz(j, _):
                rows_v[j, pl.ds(0, 16)] = zeros16
                return 0
            lax.fori_loop(wlen, _PW, padz, 0)

        def mkidx(j, _):
            v = comm_v[pl.ds(j * 16, 16)]
            sidx_v[pl.ds(j * 16, 16)] = v + cg * C
            return 0
        lax.fori_loop(0, _PW // 16, mkidx, 0)
        pltpu.sync_copy(rows_v, sum_sh.at[sidx_v], add=True)

        def mx(i, _):
            ci = comm_v[pl.ds(i, 16)][0]
            rowi = jnp.full((16,), ci, jnp.int32)
            row = rows_v[i, pl.ds(0, 16)]
            cur = plsc.load_gather(max_acc, [rowi, iota16])
            plsc.store_scatter(max_acc, [rowi, iota16], jnp.maximum(cur, row))
            return 0
        lax.fori_loop(0, wlen, mx, 0)

    def wloop(w, _):
        window(w * 784, 784)
        return 0
    lax.fori_loop(0, 7, wloop, 0)

    @pl.when(nsub < 3)
    def _():
        window(7 * 784, 768)

    @pl.when(nsub == 3)
    def _():
        window(7 * 784, 744)

    plsc.subcore_barrier()

    for qi in range(4):
        @pl.when(cg == qi)
        def _(qi=qi):
            pltpu.sync_copy(max_acc, omq[qi].at[pl.ds((c * 4 + nsub) * C, C)])

        @pl.when(jnp.logical_and(cg == qi, nsub == 0))
        def _(qi=qi):
            pltpu.sync_copy(sum_sh.at[pl.ds(qi * C, C)], osq[qi].at[pl.ds(c * C, C)])


def _pool_sc(community, hq):
    f32 = jnp.float32
    mesh = plsc.VectorSubcoreMesh(core_axis_name="c", subcore_axis_name="s")
    return pl.kernel(
        _pool_body,
        out_type=[jax.ShapeDtypeStruct((2 * C, 16), f32)] * 4 +
                 [jax.ShapeDtypeStruct((8 * C, 16), f32)] * 4,
        mesh=mesh,
        scratch_types=[
            pltpu.VMEM((800,), jnp.int32),
            pltpu.VMEM((784,), jnp.int32),
            pltpu.VMEM((784, 16), f32),
            pltpu.VMEM((C, 16), f32),
            pltpu.VMEM_SHARED((4 * C, 16), f32),
            pltpu.SemaphoreType.DMA,
        ],
        compiler_params=pltpu.CompilerParams(use_tc_tiling_on_sc=False,
                                             needs_layout_passes=False),
    )(community, *hq)


def _conv_msg_sc(gq, mq, src, dst):
    f32 = jnp.float32
    mesh = plsc.VectorSubcoreMesh(core_axis_name="c", subcore_axis_name="s")
    return pl.kernel(
        _conv_msg_body,
        out_type=[jax.ShapeDtypeStruct((N, 16), f32)] * 4,
        mesh=mesh,
        scratch_types=[
            pltpu.VMEM((_CW,), jnp.int32),
            pltpu.VMEM((_CW,), jnp.int32),
            pltpu.VMEM((_CW, 16), f32),
            pltpu.VMEM((_CW, 16), f32),
            pltpu.VMEM_SHARED((N, 16), f32),
            pltpu.SemaphoreType.DMA,
        ],
        compiler_params=pltpu.CompilerParams(use_tc_tiling_on_sc=False),
    )(*gq, *mq, src, dst)


def _rows(i):
    return (i, 0)


def _bspec(blk, width):
    return pl.BlockSpec((blk, width), _rows)


def _full(a):
    return pl.BlockSpec(a.shape, lambda i: (0, 0))


def _node1_body(x_ref, W1_ref, b1_ref, W2_ref, b2_ref, W3_ref, b3_ref,
                Wsrc_ref, Wdst_ref, bel_ref, Wn1_ref, Wr1_ref, bc1_ref,
                h_ref, asrc_ref, adst_ref, g1q0, g1q1, g1q2, g1q3, r1_ref):
    x = x_ref[...]
    x1 = jax.nn.relu(jnp.dot(x[:, :DEMO], W1_ref[...], preferred_element_type=jnp.float32) + b1_ref[...])
    x2 = jax.nn.relu(jnp.dot(x[:, DEMO:], W2_ref[...], preferred_element_type=jnp.float32) + b2_ref[...])
    h = jax.nn.relu(jnp.dot(jnp.concatenate([x1, x2], axis=1), W3_ref[...],
                            preferred_element_type=jnp.float32) + b3_ref[...])
    h_ref[...] = h
    asrc_ref[...] = jnp.dot(h, Wsrc_ref[...], preferred_element_type=jnp.float32)
    adst_ref[...] = jnp.dot(h, Wdst_ref[...], preferred_element_type=jnp.float32) + bel_ref[...]
    g1 = jnp.dot(h, Wn1_ref[...], preferred_element_type=jnp.float32)
    for q, ref in enumerate((g1q0, g1q1, g1q2, g1q3)):
        ref[...] = g1[:, q * 16:(q + 1) * 16]
    r1_ref[...] = jnp.dot(h, Wr1_ref[...], preferred_element_type=jnp.float32) + bc1_ref[...]


def _edge1_body(ea_ref, We_ref, be_ref, Wea_ref, ep_ref):
    ea = jax.nn.relu(jnp.dot(ea_ref[...], We_ref[...], preferred_element_type=jnp.float32) + be_ref[...])
    ep_ref[...] = jnp.dot(ea, Wea_ref[...], preferred_element_type=jnp.float32)


def _edge2_body(mp_ref, Wm1_ref, Wm2_ref, *out_refs):
    mask = jax.nn.sigmoid(mp_ref[...])
    m1 = jax.nn.sigmoid(jnp.dot(mask, Wm1_ref[...], preferred_element_type=jnp.float32))
    m2 = jax.nn.sigmoid(jnp.dot(mask, Wm2_ref[...], preferred_element_type=jnp.float32))
    for q in range(4):
        out_refs[q][...] = m1[:, q * 16:(q + 1) * 16]
        out_refs[4 + q][...] = m2[:, q * 16:(q + 1) * 16]


def _node2_body(s0, s1, s2, s3, cnt0_ref, cnt1_ref, r_ref, Wn2_ref, Wr2_ref, bc2_ref,
                haq0, haq1, haq2, haq3, g2q0, g2q1, g2q2, g2q3, r2_ref):
    cnt = jnp.maximum(cnt0_ref[...][:, :1] + cnt1_ref[...][:, :1], 1.0)
    s = jnp.concatenate([s0[...], s1[...], s2[...], s3[...]], axis=1)
    ha = jax.nn.relu(s / cnt + r_ref[...])
    for q, ref in enumerate((haq0, haq1, haq2, haq3)):
        ref[...] = ha[:, q * 16:(q + 1) * 16]
    g2 = jnp.dot(ha, Wn2_ref[...], preferred_element_type=jnp.float32)
    for q, ref in enumerate((g2q0, g2q1, g2q2, g2q3)):
        ref[...] = g2[:, q * 16:(q + 1) * 16]
    r2_ref[...] = jnp.dot(ha, Wr2_ref[...], preferred_element_type=jnp.float32) + bc2_ref[...]


def _node3_body(s0, s1, s2, s3, cnt0_ref, cnt1_ref, r_ref,
                hbq0, hbq1, hbq2, hbq3):
    cnt = jnp.maximum(cnt0_ref[...][:, :1] + cnt1_ref[...][:, :1], 1.0)
    s = jnp.concatenate([s0[...], s1[...], s2[...], s3[...]], axis=1)
    hb = jax.nn.relu(s / cnt + r_ref[...])
    for q, ref in enumerate((hbq0, hbq1, hbq2, hbq3)):
        ref[...] = hb[:, q * 16:(q + 1) * 16]


def _final_body(*refs):
    pc0, pc1 = refs[80], refs[81]
    Wl1_ref, bl1_ref, Wl2_ref, bl2_ref, out_ref = refs[82:]
    cnt = pc0[...][:, :1] + pc1[...][:, :1]
    cmax = jnp.maximum(cnt, 1.0)

    def build(rs):
        sq, mq = rs[0:8], rs[8:40]
        means = [(sq[2 * q][...] + sq[2 * q + 1][...]) / cmax for q in range(4)]
        maxs = []
        for q in range(4):
            m = mq[8 * q][...]
            for k in range(1, 8):
                m = jnp.maximum(m, mq[8 * q + k][...])
            maxs.append(jnp.where(cnt > 0, m, 0.0))
        return jnp.concatenate(means + maxs, axis=1)

    z = build(refs[0:40]) + build(refs[40:80])
    z = jax.nn.relu(jnp.dot(z, Wl1_ref[...], preferred_element_type=jnp.float32) + bl1_ref[...])
    out_ref[...] = jnp.dot(z, Wl2_ref[...], preferred_element_type=jnp.float32) + bl2_ref[...]


def kernel(x, edge_index, edge_attr, community, multi_community_nodes, multi_community_index,
           We, be, W1, b1, W2, b2, W3, b3, Wel, bel,
           Wn1, Wr1, Wm1, bc1, Wn2, Wr2, Wm2, bc2, Wl1, bl1, Wl2, bl2):
    src, dst = edge_index[0], edge_index[1]
    Wsrc, Wdst, Wea = Wel[:32], Wel[32:64], Wel[64:]

    NBLK = 2000
    EBLK = 4000
    f32 = jnp.float32

    (h, asrc, adst, g1q0, g1q1, g1q2, g1q3, r1) = pl.pallas_call(
        _node1_body,
        grid=(N // NBLK,),
        in_specs=[_bspec(NBLK, 20)] + [_full(a) for a in
                  (W1, b1[None], W2, b2[None], W3, b3[None], Wsrc, Wdst, bel[None],
                   Wn1, Wr1, bc1[None])],
        out_specs=[_bspec(NBLK, 32), _bspec(NBLK, 32), _bspec(NBLK, 32)] +
                  [_bspec(NBLK, 16)] * 4 + [_bspec(NBLK, 64)],
        out_shape=[jax.ShapeDtypeStruct((N, 32), f32)] * 3 +
                  [jax.ShapeDtypeStruct((N, 16), f32)] * 4 +
                  [jax.ShapeDtypeStruct((N, 64), f32)],
    )(x, W1, b1[None], W2, b2[None], W3, b3[None], Wsrc, Wdst, bel[None], Wn1, Wr1, bc1[None])

    e_part = pl.pallas_call(
        _edge1_body,
        grid=(E // EBLK,),
        in_specs=[_bspec(EBLK, 4), _full(We), _full(be[None]), _full(Wea)],
        out_specs=_bspec(EBLK, 32),
        out_shape=jax.ShapeDtypeStruct((E, 32), f32),
    )(edge_attr, We, be[None], Wea)

    mask_pre = _mask_sc(asrc, adst, e_part, src, dst)
    cnt0, cnt1, pc0, pc1 = _hist_sc(dst, community)

    m1q = [None] * 4
    m2q = [None] * 4
    (m1q[0], m1q[1], m1q[2], m1q[3], m2q[0], m2q[1], m2q[2], m2q[3]) = pl.pallas_call(
        _edge2_body,
        grid=(E // EBLK,),
        in_specs=[_bspec(EBLK, 32), _full(Wm1), _full(Wm2)],
        out_specs=[_bspec(EBLK, 16)] * 8,
        out_shape=[jax.ShapeDtypeStruct((E, 16), f32)] * 8,
    )(mask_pre, Wm1, Wm2)

    s1q = _conv_msg_sc((g1q0, g1q1, g1q2, g1q3), m1q, src, dst)

    (haq0, haq1, haq2, haq3, g2q0, g2q1, g2q2, g2q3, r2) = pl.pallas_call(
        _node2_body,
        grid=(N // NBLK,),
        in_specs=[_bspec(NBLK, 16)] * 4 + [_bspec(NBLK, 16), _bspec(NBLK, 16), _bspec(NBLK, 64),
                  _full(Wn2), _full(Wr2), _full(bc2[None])],
        out_specs=[_bspec(NBLK, 16)] * 8 + [_bspec(NBLK, 64)],
        out_shape=[jax.ShapeDtypeStruct((N, 16), f32)] * 8 +
                  [jax.ShapeDtypeStruct((N, 64), f32)],
    )(*s1q, cnt0, cnt1, r1, Wn2, Wr2, bc2[None])

    s2q = _conv_msg_sc((g2q0, g2q1, g2q2, g2q3), m2q, src, dst)

    hbq = pl.pallas_call(
        _node3_body,
        grid=(N // NBLK,),
        in_specs=[_bspec(NBLK, 16)] * 4 + [_bspec(NBLK, 16), _bspec(NBLK, 16), _bspec(NBLK, 64)],
        out_specs=[_bspec(NBLK, 16)] * 4,
        out_shape=[jax.ShapeDtypeStruct((N, 16), f32)] * 4,
    )(*s2q, cnt0, cnt1, r2)

    pa = _pool_sc(community, (haq0, haq1, haq2, haq3))
    pb = _pool_sc(community, hbq)

    CBLK = 200
    nb = C // CBLK

    def _row_spec(k):
        return pl.BlockSpec((CBLK, 16), lambda i, k=k: (k * nb + i, 0))

    def expand(p):
        sq, mq = p[0:4], p[4:8]
        args, specs = [], []
        for q in range(4):
            for k in range(2):
                args.append(sq[q]); specs.append(_row_spec(k))
        for q in range(4):
            for k in range(8):
                args.append(mq[q]); specs.append(_row_spec(k))
        return args, specs

    args_a, specs_a = expand(pa)
    args_b, specs_b = expand(pb)
    out = pl.pallas_call(
        _final_body,
        grid=(nb,),
        in_specs=specs_a + specs_b + [_row_spec(0), _row_spec(0)] +
                 [_full(Wl1), _full(bl1[None]), _full(Wl2), _full(bl2[None])],
        out_specs=_bspec(CBLK, 1),
        out_shape=jax.ShapeDtypeStruct((C, 1), f32),
    )(*args_a, *args_b, pc0, pc1, Wl1, bl1[None], Wl2, bl2[None])
    return out[:, 0]
